# baseline (speedup 1.0000x reference)
import functools

import jax
import jax.numpy as jnp
from jax import lax
from jax.experimental import pallas as pl
from jax.experimental.pallas import tpu as pltpu
from jax.experimental.pallas import tpu_sc as plsc

BATCH = 16384
DIM = 32
LANES = 16
WIN = 128
PHASE = 4
NSUB = LANES // PHASE

_info = plsc.get_sparse_core_info()
NC = _info.num_cores
NS = _info.num_subcores
NW = NC * NS
B_PER_W = BATCH // NW
NGROUP = B_PER_W // LANES
IDX_PAD = B_PER_W + LANES


def _bpr_body(u_hbm, i_hbm, j_hbm, ut_hbm, it_hbm, pos_hbm, neg_hbm,
              idx_u, idx_i, idx_j, wu, wi, wj, pos_v, neg_v, sem0, sem1):
    sems = (sem0, sem1)
    wid = lax.axis_index("s") * NC + lax.axis_index("c")
    base = wid * B_PER_W

    pltpu.sync_copy(u_hbm.at[pl.ds(base, B_PER_W)], idx_u.at[pl.ds(0, B_PER_W)])
    pltpu.sync_copy(i_hbm.at[pl.ds(base, B_PER_W)], idx_i.at[pl.ds(0, B_PER_W)])
    pltpu.sync_copy(j_hbm.at[pl.ds(base, B_PER_W)], idx_j.at[pl.ds(0, B_PER_W)])
    zeros16 = jnp.zeros((LANES,), jnp.int32)
    idx_u[pl.ds(B_PER_W, LANES)] = zeros16
    idx_i[pl.ds(B_PER_W, LANES)] = zeros16
    idx_j[pl.ds(B_PER_W, LANES)] = zeros16

    lanes = lax.iota(jnp.int32, LANES)
    slot = lanes & (PHASE - 1)

    def fire_sub(vu, vi, vj, s, buf):
        sem = sems[buf]
        for t in range(PHASE):
            k = s * PHASE + t
            ou = pl.multiple_of((vu[k] >> 7) * WIN, WIN)
            oi = pl.multiple_of((vi[k] >> 7) * WIN, WIN)
            oj = pl.multiple_of((vj[k] >> 7) * WIN, WIN)
            pltpu.async_copy(ut_hbm.at[:, pl.ds(ou, WIN)], wu.at[buf, t], sem)
            pltpu.async_copy(it_hbm.at[:, pl.ds(oi, WIN)], wi.at[buf, t], sem)
            pltpu.async_copy(it_hbm.at[:, pl.ds(oj, WIN)], wj.at[buf, t], sem)

    def drain_sub(buf):
        sem = sems[buf]
        src = ut_hbm.at[:, pl.ds(0, WIN)]
        for t in range(PHASE):
            pltpu.make_async_copy(src, wu.at[buf, t], sem).wait()
            pltpu.make_async_copy(src, wi.at[buf, t], sem).wait()
            pltpu.make_async_copy(src, wj.at[buf, t], sem).wait()

    def extract_sub(rl_u, rl_i, rl_j, s, buf):
        perm = s * PHASE + slot
        ru = rl_u.at[perm].get(mode="promise_in_bounds")
        ri = rl_i.at[perm].get(mode="promise_in_bounds")
        rj = rl_j.at[perm].get(mode="promise_in_bounds")
        accp = jnp.zeros((LANES,), jnp.float32)
        accn = jnp.zeros((LANES,), jnp.float32)
        wub = wu.at[buf]
        wib = wi.at[buf]
        wjb = wj.at[buf]
        for c in range(DIM):
            cvec = jnp.full((LANES,), c, jnp.int32)
            gu = plsc.load_gather(wub, [slot, cvec, ru])
            gi = plsc.load_gather(wib, [slot, cvec, ri])
            gj = plsc.load_gather(wjb, [slot, cvec, rj])
            accp = accp + gu * gi
            accn = accn + gu * gj
        return accp, accn

    def load_idx(g):
        goff = g * LANES
        return (
            idx_u[pl.ds(goff, LANES)],
            idx_i[pl.ds(goff, LANES)],
            idx_j[pl.ds(goff, LANES)],
        )

    vu0, vi0, vj0 = load_idx(0)
    fire_sub(vu0, vi0, vj0, 0, 0)

    def group_body(g, carry):
        vu, vi, vj = load_idx(g)
        vun, vin, vjn = load_idx(g + 1)
        rl_u = vu & (WIN - 1)
        rl_i = vi & (WIN - 1)
        rl_j = vj & (WIN - 1)

        accp = jnp.zeros((LANES,), jnp.float32)
        accn = jnp.zeros((LANES,), jnp.float32)
        for s in range(NSUB):
            buf = s & 1
            nbuf = (s + 1) & 1
            if s + 1 < NSUB:
                fire_sub(vu, vi, vj, s + 1, nbuf)
            else:
                fire_sub(vun, vin, vjn, 0, nbuf)
            drain_sub(buf)
            p, n = extract_sub(rl_u, rl_i, rl_j, s, buf)
            m = (lanes >> 2) == s
            accp = jnp.where(m, p, accp)
            accn = jnp.where(m, n, accn)

        goff = g * LANES
        pos_v[pl.ds(goff, LANES)] = accp
        neg_v[pl.ds(goff, LANES)] = accn
        return carry

    lax.fori_loop(0, NGROUP, group_body, 0)

    drain_sub(0)

    pltpu.sync_copy(pos_v, pos_hbm.at[pl.ds(base, B_PER_W)])
    pltpu.sync_copy(neg_v, neg_hbm.at[pl.ds(base, B_PER_W)])


@jax.jit
def _bpr_call(u, i, j, user_table, item_table):
    ut_t = user_table.T
    it_t = item_table.T
    mesh = plsc.VectorSubcoreMesh(core_axis_name="c", subcore_axis_name="s")
    f = functools.partial(
        pl.kernel,
        mesh=mesh,
        compiler_params=pltpu.CompilerParams(needs_layout_passes=False),
        out_type=[
            jax.ShapeDtypeStruct((BATCH,), jnp.float32),
            jax.ShapeDtypeStruct((BATCH,), jnp.float32),
        ],
        scratch_types=[
            pltpu.VMEM((IDX_PAD,), jnp.int32),
            pltpu.VMEM((IDX_PAD,), jnp.int32),
            pltpu.VMEM((IDX_PAD,), jnp.int32),
            pltpu.VMEM((2, PHASE, DIM, WIN), jnp.float32),
            pltpu.VMEM((2, PHASE, DIM, WIN), jnp.float32),
            pltpu.VMEM((2, PHASE, DIM, WIN), jnp.float32),
            pltpu.VMEM((B_PER_W,), jnp.float32),
            pltpu.VMEM((B_PER_W,), jnp.float32),
            pltpu.SemaphoreType.DMA,
            pltpu.SemaphoreType.DMA,
        ],
    )(_bpr_body)
    return f(u, i, j, ut_t, it_t)


def kernel(u, i, j, user_table, item_table):
    u = u.astype(jnp.int32)
    i = i.astype(jnp.int32)
    j = j.astype(jnp.int32)
    pos, neg = _bpr_call(u, i, j, user_table, item_table)
    return (pos, neg)


# speedup vs baseline: 1.0959x; 1.0959x over previous
import functools

import jax
import jax.numpy as jnp
from jax import lax
from jax.experimental import pallas as pl
from jax.experimental.pallas import tpu as pltpu
from jax.experimental.pallas import tpu_sc as plsc

BATCH = 16384
DIM = 32
LANES = 16
WIN = 128
PHASE = 8

_info = plsc.get_sparse_core_info()
NC = _info.num_cores
NS = _info.num_subcores
NW = NC * NS
B_PER_W = BATCH // NW
NGROUP = B_PER_W // LANES


def _bpr_body(u_hbm, i_hbm, j_hbm, ut_hbm, it_hbm, pos_hbm, neg_hbm,
              idx_u, idx_i, idx_j, wu, wi, wj, pos_v, neg_v, sem):
    wid = lax.axis_index("s") * NC + lax.axis_index("c")
    base = wid * B_PER_W

    pltpu.sync_copy(u_hbm.at[pl.ds(base, B_PER_W)], idx_u)
    pltpu.sync_copy(i_hbm.at[pl.ds(base, B_PER_W)], idx_i)
    pltpu.sync_copy(j_hbm.at[pl.ds(base, B_PER_W)], idx_j)

    lanes = lax.iota(jnp.int32, LANES)
    slot = lanes & (PHASE - 1)

    def fire_phase(vu, vi, vj, ph):
        for t in range(PHASE):
            k = ph * PHASE + t
            ou = pl.multiple_of((vu[k] >> 7) * WIN, WIN)
            oi = pl.multiple_of((vi[k] >> 7) * WIN, WIN)
            oj = pl.multiple_of((vj[k] >> 7) * WIN, WIN)
            for cb in range(DIM // 8):
                dcb = pl.ds(cb * 8, 8)
                pltpu.async_copy(
                    ut_hbm.at[dcb, pl.ds(ou, WIN)], wu.at[t, dcb], sem)
                pltpu.async_copy(
                    it_hbm.at[dcb, pl.ds(oi, WIN)], wi.at[t, dcb], sem)
                pltpu.async_copy(
                    it_hbm.at[dcb, pl.ds(oj, WIN)], wj.at[t, dcb], sem)

    def drain_phase():
        for t in range(PHASE):
            for cb in range(DIM // 8):
                dcb = pl.ds(cb * 8, 8)
                src = ut_hbm.at[dcb, pl.ds(0, WIN)]
                pltpu.make_async_copy(src, wu.at[t, dcb], sem).wait()
                pltpu.make_async_copy(src, wi.at[t, dcb], sem).wait()
                pltpu.make_async_copy(src, wj.at[t, dcb], sem).wait()

    def extract_phase(rl_u, rl_i, rl_j, ph):
        perm = ph * PHASE + slot
        ru = rl_u.at[perm].get(mode="promise_in_bounds")
        ri = rl_i.at[perm].get(mode="promise_in_bounds")
        rj = rl_j.at[perm].get(mode="promise_in_bounds")
        accp = jnp.zeros((LANES,), jnp.float32)
        accn = jnp.zeros((LANES,), jnp.float32)
        for c in range(DIM):
            cvec = jnp.full((LANES,), c, jnp.int32)
            gu = plsc.load_gather(wu, [slot, cvec, ru])
            gi = plsc.load_gather(wi, [slot, cvec, ri])
            gj = plsc.load_gather(wj, [slot, cvec, rj])
            accp = accp + gu * gi
            accn = accn + gu * gj
        return accp, accn

    def group_body(g, carry):
        goff = g * LANES
        vu = idx_u[pl.ds(goff, LANES)]
        vi = idx_i[pl.ds(goff, LANES)]
        vj = idx_j[pl.ds(goff, LANES)]
        rl_u = vu & (WIN - 1)
        rl_i = vi & (WIN - 1)
        rl_j = vj & (WIN - 1)

        fire_phase(vu, vi, vj, 0)
        drain_phase()
        p0, n0 = extract_phase(rl_u, rl_i, rl_j, 0)
        fire_phase(vu, vi, vj, 1)
        drain_phase()
        p1, n1 = extract_phase(rl_u, rl_i, rl_j, 1)

        lo = lanes < PHASE
        pos_v[pl.ds(goff, LANES)] = jnp.where(lo, p0, p1)
        neg_v[pl.ds(goff, LANES)] = jnp.where(lo, n0, n1)
        return carry

    lax.fori_loop(0, NGROUP, group_body, 0)

    pltpu.sync_copy(pos_v, pos_hbm.at[pl.ds(base, B_PER_W)])
    pltpu.sync_copy(neg_v, neg_hbm.at[pl.ds(base, B_PER_W)])


@jax.jit
def _bpr_call(u, i, j, user_table, item_table):
    ut_t = user_table.T
    it_t = item_table.T
    mesh = plsc.VectorSubcoreMesh(core_axis_name="c", subcore_axis_name="s")
    f = functools.partial(
        pl.kernel,
        mesh=mesh,
        compiler_params=pltpu.CompilerParams(needs_layout_passes=False),
        out_type=[
            jax.ShapeDtypeStruct((BATCH,), jnp.float32),
            jax.ShapeDtypeStruct((BATCH,), jnp.float32),
        ],
        scratch_types=[
            pltpu.VMEM((B_PER_W,), jnp.int32),
            pltpu.VMEM((B_PER_W,), jnp.int32),
            pltpu.VMEM((B_PER_W,), jnp.int32),
            pltpu.VMEM((PHASE, DIM, WIN), jnp.float32),
            pltpu.VMEM((PHASE, DIM, WIN), jnp.float32),
            pltpu.VMEM((PHASE, DIM, WIN), jnp.float32),
            pltpu.VMEM((B_PER_W,), jnp.float32),
            pltpu.VMEM((B_PER_W,), jnp.float32),
            pltpu.SemaphoreType.DMA,
        ],
    )(_bpr_body)
    return f(u, i, j, ut_t, it_t)


def kernel(u, i, j, user_table, item_table):
    u = u.astype(jnp.int32)
    i = i.astype(jnp.int32)
    j = j.astype(jnp.int32)
    pos, neg = _bpr_call(u, i, j, user_table, item_table)
    return (pos, neg)
